# Initial kernel scaffold; baseline (speedup 1.0000x reference)
import jax
import jax.numpy as jnp
from jax.experimental import pallas as pl


def kernel(x, edge_index, edge_weight):
    raise NotImplementedError("write your pallas kernel here")


# speedup vs baseline: 1.5125x; 1.5125x over previous
import jax
import jax.numpy as jnp
from jax import lax
from jax.experimental import pallas as pl
from jax.experimental.pallas import tpu as pltpu
from jax.experimental.pallas import tpu_sc as plsc

_N = 10000
_E = 160000
_D = 256
_MIN_NORM = 1e-15
_MAX_TANH = 15.0

_NC = 2
_NS = 16
_L = 16
_NW = _NC * _NS
_B = 320
_NPAD = _NW * _B
_C = 2000
_NCHK = _E // _C
_GC = _C // _L
_G = 80
_P = 2176


def _logmap0_body(x_ref, o_ref):
    x = x_ref[...]
    norm = jnp.sqrt(jnp.sum(x * x, axis=1, keepdims=True))
    p_norm = jnp.clip(norm, _MIN_NORM, None)
    z = jnp.clip(p_norm, -1.0 + 1e-7, 1.0 - 1e-7)
    artanh = 0.5 * jnp.log((1.0 + z) / (1.0 - z))
    o_ref[...] = x * (artanh / p_norm)


def _expmap0_proj_body(u_ref, o_ref):
    u = u_ref[...]
    norm = jnp.sqrt(jnp.sum(u * u, axis=1, keepdims=True))
    u_norm = jnp.clip(norm, _MIN_NORM, None)
    res = jnp.tanh(jnp.clip(u_norm, -_MAX_TANH, _MAX_TANH)) * u / u_norm
    rnorm = jnp.sqrt(jnp.sum(res * res, axis=1, keepdims=True))
    rnorm = jnp.clip(rnorm, _MIN_NORM, None)
    maxnorm = 1.0 - 4e-3
    o_ref[...] = jnp.where(rnorm > maxnorm, res / rnorm * maxnorm, res)


def _rowmap_call(body, n_rows, block_rows):
    grid = n_rows // block_rows
    spec = pl.BlockSpec((block_rows, _D), lambda i: (i, 0))
    return pl.pallas_call(
        body,
        grid=(grid,),
        in_specs=[spec],
        out_specs=spec,
        out_shape=jax.ShapeDtypeStruct((n_rows, _D), jnp.float32),
    )


def _sc_agg_body(xt_hbm, dst_hbm, src_hbm, w_hbm, out_hbm,
                 dst_v, src_v, w_v, p_src, p_dloc, p_w, rows_v, acc, gsem):
    c = lax.axis_index("c")
    s = lax.axis_index("s")
    wid = s * _NC + c
    lo = pl.multiple_of(wid * _B, 8)
    hi = lo + _B
    iota = lax.iota(jnp.int32, _L)
    zf = jnp.zeros((_L,), jnp.float32)
    zi = jnp.zeros((_L,), jnp.int32)

    def zrow(r, _):
        for k in range(_D // _L):
            acc[r, pl.ds(k * _L, _L)] = zf
        return 0

    lax.fori_loop(0, _B, zrow, 0)

    def zpend(j, _):
        at = pl.ds(pl.multiple_of(j * _L, _L), _L)
        p_src[at] = zi
        p_dloc[at] = zi
        p_w[at] = zf
        return 0

    lax.fori_loop(0, _P // _L, zpend, 0)

    def fire_window(woff):
        woff = pl.multiple_of(woff, 16)
        pltpu.async_copy(
            xt_hbm.at[p_src.at[pl.ds(woff, _G)]], rows_v, gsem
        ).wait()

        def edge(e, _):
            esplat = zi + e + woff
            wv = plsc.load_gather(p_w, [esplat])
            dl = plsc.load_gather(p_dloc, [esplat])
            er = zi + e
            for k in range(_D // _L):
                col = iota + (k * _L)
                v = plsc.load_gather(rows_v, [er, col])
                plsc.addupdate_scatter(acc, [dl, col], v * wv)
            return 0

        lax.fori_loop(0, _G, edge, 0)

    def chunk_body(g, pend0):
        base = g * _C
        pltpu.sync_copy(dst_hbm.at[pl.ds(base, _C)], dst_v)
        pltpu.sync_copy(src_hbm.at[pl.ds(base, _C)], src_v)
        pltpu.sync_copy(w_hbm.at[pl.ds(base, _C)], w_v)

        def grp(i, pend):
            sl = pl.ds(i * _L, _L)
            dstv = dst_v[sl]
            mask = (dstv >= lo) & (dstv < hi)
            at = pl.ds(pend, _L)
            plsc.store_compressed(p_src.at[at], src_v[sl], mask=mask)
            plsc.store_compressed(p_dloc.at[at], dstv - lo, mask=mask)
            plsc.store_compressed(p_w.at[at], w_v[sl], mask=mask)
            return pend + plsc.all_reduce_population_count(mask)[0]

        pend = lax.fori_loop(0, _GC, grp, pend0)

        def wcond(carry):
            return carry[0] + _G <= carry[1]

        def wbody(carry):
            woff, pend = carry
            fire_window(woff)
            return (woff + _G, pend)

        woff, pend = lax.while_loop(wcond, wbody, (0, pend))

        woff = pl.multiple_of(woff, 16)
        for j in range(_G // _L + 1):
            sl_to = pl.ds(j * _L, _L)
            sl_from = pl.ds(woff + j * _L, _L)
            p_src[sl_to] = p_src[sl_from]
            p_dloc[sl_to] = p_dloc[sl_from]
            p_w[sl_to] = p_w[sl_from]
        return pend - woff

    pend = lax.fori_loop(0, _NCHK, chunk_body, 0)

    a = pl.multiple_of((pend // _L) * _L, _L)
    pw = p_w[pl.ds(a, _L)]
    p_w[pl.ds(a, _L)] = jnp.where(iota < pend - a, pw, 0.0)
    for j in range(1, _G // _L + 1):
        p_w[pl.ds(a + j * _L, _L)] = zf
    fire_window(jnp.int32(0))

    pltpu.sync_copy(acc, out_hbm.at[pl.ds(lo, _B)])


def _sc_agg(xt, dst, src, w):
    call = pl.kernel(
        _sc_agg_body,
        out_type=jax.ShapeDtypeStruct((_NPAD, _D), jnp.float32),
        mesh=plsc.VectorSubcoreMesh(core_axis_name="c", subcore_axis_name="s",
                                    num_cores=_NC, num_subcores=_NS),
        compiler_params=pltpu.CompilerParams(needs_layout_passes=False),
        scratch_types=[
            pltpu.VMEM((_C,), jnp.int32),
            pltpu.VMEM((_C,), jnp.int32),
            pltpu.VMEM((_C,), jnp.float32),
            pltpu.VMEM((_P,), jnp.int32),
            pltpu.VMEM((_P,), jnp.int32),
            pltpu.VMEM((_P,), jnp.float32),
            pltpu.VMEM((_G, _D), jnp.float32),
            pltpu.VMEM((_B, _D), jnp.float32),
            pltpu.SemaphoreType.DMA,
        ],
    )
    return call(xt, dst, src, w)


def kernel(x, edge_index, edge_weight):
    x_tangent = _rowmap_call(_logmap0_body, _N, 400)(x)
    dst = edge_index[0]
    src = edge_index[1]
    support_t = _sc_agg(x_tangent, dst, src, edge_weight)
    return _rowmap_call(_expmap0_proj_body, _N, 400)(support_t)


# speedup vs baseline: 1.6578x; 1.0961x over previous
import jax
import jax.numpy as jnp
from jax import lax
from jax.experimental import pallas as pl
from jax.experimental.pallas import tpu as pltpu
from jax.experimental.pallas import tpu_sc as plsc

_N = 10000
_E = 160000
_D = 256
_MIN_NORM = 1e-15
_MAX_TANH = 15.0

_NC = 2
_NS = 16
_L = 16
_NW = _NC * _NS
_B = 320
_NPAD = _NW * _B
_C = 2000
_NCHK = _E // _C
_GC = _C // _L
_G = 80
_P = 2176


def _logmap0_body(x_ref, o_ref):
    x = x_ref[...]
    norm = jnp.sqrt(jnp.sum(x * x, axis=1, keepdims=True))
    p_norm = jnp.clip(norm, _MIN_NORM, None)
    z = jnp.clip(p_norm, -1.0 + 1e-7, 1.0 - 1e-7)
    artanh = 0.5 * jnp.log((1.0 + z) / (1.0 - z))
    o_ref[...] = x * (artanh / p_norm)


def _expmap0_proj_body(u_ref, o_ref):
    u = u_ref[...]
    norm = jnp.sqrt(jnp.sum(u * u, axis=1, keepdims=True))
    u_norm = jnp.clip(norm, _MIN_NORM, None)
    res = jnp.tanh(jnp.clip(u_norm, -_MAX_TANH, _MAX_TANH)) * u / u_norm
    rnorm = jnp.sqrt(jnp.sum(res * res, axis=1, keepdims=True))
    rnorm = jnp.clip(rnorm, _MIN_NORM, None)
    maxnorm = 1.0 - 4e-3
    o_ref[...] = jnp.where(rnorm > maxnorm, res / rnorm * maxnorm, res)


def _rowmap_call(body, n_rows, block_rows):
    grid = n_rows // block_rows
    spec = pl.BlockSpec((block_rows, _D), lambda i: (i, 0))
    return pl.pallas_call(
        body,
        grid=(grid,),
        in_specs=[spec],
        out_specs=spec,
        out_shape=jax.ShapeDtypeStruct((n_rows, _D), jnp.float32),
    )


def _sc_agg_body(xt_hbm, pk_hbm, out_hbm,
                 pk0, pk1, p_src, p_dloc, p_w, rows_v, acc,
                 gsem, esem0, esem1):
    c = lax.axis_index("c")
    s = lax.axis_index("s")
    wid = s * _NC + c
    lo = pl.multiple_of(wid * _B, 8)
    hi = lo + _B
    iota = lax.iota(jnp.int32, _L)
    iota3 = iota * 3
    zf = jnp.zeros((_L,), jnp.float32)
    zi = jnp.zeros((_L,), jnp.int32)

    def zrow(r, _):
        for k in range(_D // _L):
            acc[r, pl.ds(k * _L, _L)] = zf
        return 0

    lax.fori_loop(0, _B, zrow, 0)

    def zpend(j, _):
        at = pl.ds(pl.multiple_of(j * _L, _L), _L)
        p_src[at] = zi
        p_dloc[at] = zi
        p_w[at] = zf
        return 0

    lax.fori_loop(0, _P // _L, zpend, 0)

    def fire_window(woff):
        woff = pl.multiple_of(woff, 16)
        pltpu.async_copy(
            xt_hbm.at[p_src.at[pl.ds(woff, _G)]], rows_v, gsem
        ).wait()

        def edge(e, _):
            esplat = zi + e + woff
            wv = plsc.load_gather(p_w, [esplat])
            dl = plsc.load_gather(p_dloc, [esplat])
            er = zi + e
            for k in range(_D // _L):
                col = iota + (k * _L)
                v = plsc.load_gather(rows_v, [er, col])
                plsc.addupdate_scatter(acc, [dl, col], v * wv)
            return 0

        lax.fori_loop(0, _G, edge, 0)

    def process_chunk(pk, pend0):
        def grp(i, pend):
            base3 = i * (3 * _L)
            dstv = plsc.load_gather(pk, [iota3 + base3])
            mask = (dstv >= lo) & (dstv < hi)
            srcv = plsc.load_gather(pk, [iota3 + (base3 + 1)])
            wv = plsc.bitcast(plsc.load_gather(pk, [iota3 + (base3 + 2)]),
                              jnp.float32)
            at = pl.ds(pend, _L)
            plsc.store_compressed(p_src.at[at], srcv, mask=mask)
            plsc.store_compressed(p_dloc.at[at], dstv - lo, mask=mask)
            plsc.store_compressed(p_w.at[at], wv, mask=mask)
            return pend + plsc.all_reduce_population_count(mask)[0]

        pend = lax.fori_loop(0, _GC, grp, pend0)

        def wcond(carry):
            return carry[0] + _G <= carry[1]

        def wbody(carry):
            woff, pend = carry
            fire_window(woff)
            return (woff + _G, pend)

        woff, pend = lax.while_loop(wcond, wbody, (0, pend))

        woff = pl.multiple_of(woff, 16)
        for j in range(_G // _L + 1):
            sl_to = pl.ds(j * _L, _L)
            sl_from = pl.ds(woff + j * _L, _L)
            p_src[sl_to] = p_src[sl_from]
            p_dloc[sl_to] = p_dloc[sl_from]
            p_w[sl_to] = p_w[sl_from]
        return pend - woff

    c3 = 3 * _C
    pltpu.async_copy(pk_hbm.at[pl.ds(0, c3)], pk0, esem0)
    pltpu.async_copy(pk_hbm.at[pl.ds(c3, c3)], pk1, esem1)

    def pair_body(h, pend):
        g0 = 2 * h
        pltpu.make_async_copy(pk_hbm.at[pl.ds(0, c3)], pk0, esem0).wait()
        pend = process_chunk(pk0, pend)

        @pl.when(g0 + 2 < _NCHK)
        def _():
            pltpu.async_copy(
                pk_hbm.at[pl.ds(pl.multiple_of((g0 + 2) * c3, 8), c3)],
                pk0, esem0)

        pltpu.make_async_copy(pk_hbm.at[pl.ds(0, c3)], pk1, esem1).wait()
        pend = process_chunk(pk1, pend)

        @pl.when(g0 + 3 < _NCHK)
        def _():
            pltpu.async_copy(
                pk_hbm.at[pl.ds(pl.multiple_of((g0 + 3) * c3, 8), c3)],
                pk1, esem1)

        return pend

    pend = lax.fori_loop(0, _NCHK // 2, pair_body, 0)

    a = pl.multiple_of((pend // _L) * _L, _L)
    pw = p_w[pl.ds(a, _L)]
    p_w[pl.ds(a, _L)] = jnp.where(iota < pend - a, pw, 0.0)
    for j in range(1, _G // _L + 1):
        p_w[pl.ds(a + j * _L, _L)] = zf
    fire_window(jnp.int32(0))

    pltpu.sync_copy(acc, out_hbm.at[pl.ds(lo, _B)])


def _sc_agg(xt, pk):
    call = pl.kernel(
        _sc_agg_body,
        out_type=jax.ShapeDtypeStruct((_NPAD, _D), jnp.float32),
        mesh=plsc.VectorSubcoreMesh(core_axis_name="c", subcore_axis_name="s",
                                    num_cores=_NC, num_subcores=_NS),
        compiler_params=pltpu.CompilerParams(needs_layout_passes=False),
        scratch_types=[
            pltpu.VMEM((3 * _C,), jnp.int32),
            pltpu.VMEM((3 * _C,), jnp.int32),
            pltpu.VMEM((_P,), jnp.int32),
            pltpu.VMEM((_P,), jnp.int32),
            pltpu.VMEM((_P,), jnp.float32),
            pltpu.VMEM((_G, _D), jnp.float32),
            pltpu.VMEM((_B, _D), jnp.float32),
            pltpu.SemaphoreType.DMA,
            pltpu.SemaphoreType.DMA,
            pltpu.SemaphoreType.DMA,
        ],
    )
    return call(xt, pk)


def kernel(x, edge_index, edge_weight):
    x_tangent = _rowmap_call(_logmap0_body, _N, 400)(x)
    wbits = jax.lax.bitcast_convert_type(edge_weight, jnp.int32)
    packed = jnp.stack(
        [edge_index[0], edge_index[1], wbits], axis=1).reshape(-1)
    support_t = _sc_agg(x_tangent, packed)
    return _rowmap_call(_expmap0_proj_body, _N, 400)(support_t)
